# baseline (speedup 1.0000x reference)
import functools

import jax
import jax.numpy as jnp
from jax import lax
from jax.experimental import pallas as pl
from jax.experimental.pallas import tpu as pltpu
from jax.experimental.pallas import tpu_sc as plsc

D_MODEL = 128
BATCH = 16384

_info = plsc.get_sparse_core_info()
_NC, _NS = _info.num_cores, _info.num_subcores
_NW = _NC * _NS
_B_PER_W = BATCH // _NW

_mesh = plsc.VectorSubcoreMesh(core_axis_name="c", subcore_axis_name="s")

_NCHUNK = 4
_CH = _B_PER_W // _NCHUNK


@functools.partial(
    pl.kernel,
    mesh=_mesh,
    out_type=(
        jax.ShapeDtypeStruct((BATCH, D_MODEL), jnp.float32),
        jax.ShapeDtypeStruct((BATCH,), jnp.float32),
    ),
    scratch_types=[
        pltpu.VMEM((_B_PER_W,), jnp.int32),
        pltpu.VMEM((_B_PER_W, D_MODEL), jnp.float32),
        pltpu.VMEM((_B_PER_W,), jnp.float32),
    ]
    + [pltpu.SemaphoreType.DMA] * (4 * _NCHUNK + 1),
)
def _gather_rows(weight_hbm, bias_hbm, idx_hbm, out_w_hbm, out_b_hbm,
                 idx_v, rows_v, bvals_v, *sems):
    wid = lax.axis_index("s") * _NC + lax.axis_index("c")
    base = wid * _B_PER_W
    idx_cps = [
        pltpu.async_copy(
            idx_hbm.at[pl.ds(base + c * _CH, _CH)],
            idx_v.at[pl.ds(c * _CH, _CH)],
            sems[3 * _NCHUNK + c],
        )
        for c in range(_NCHUNK)
    ]
    gathers = []
    bgathers = []
    for c in range(_NCHUNK):
        idx_cps[c].wait()
        gathers.append(
            pltpu.async_copy(
                weight_hbm.at[idx_v.at[pl.ds(c * _CH, _CH)]],
                rows_v.at[pl.ds(c * _CH, _CH)],
                sems[c],
            )
        )
        bgathers.append(
            pltpu.async_copy(
                bias_hbm.at[idx_v.at[pl.ds(c * _CH, _CH)]],
                bvals_v.at[pl.ds(c * _CH, _CH)],
                sems[2 * _NCHUNK + c],
            )
        )
    writes = []
    for c in range(_NCHUNK):
        gathers[c].wait()
        writes.append(
            pltpu.async_copy(
                rows_v.at[pl.ds(c * _CH, _CH)],
                out_w_hbm.at[pl.ds(base + c * _CH, _CH)],
                sems[_NCHUNK + c],
            )
        )
    for c in range(_NCHUNK):
        bgathers[c].wait()
    writes.append(
        pltpu.async_copy(
            bvals_v, out_b_hbm.at[pl.ds(base, _B_PER_W)], sems[4 * _NCHUNK]
        )
    )
    for w in writes:
        w.wait()


def kernel(coords, voxel_indices, weight, bias):
    del coords
    idx = voxel_indices.astype(jnp.int32)
    return _gather_rows(weight, bias, idx)


# speedup vs baseline: 1.0312x; 1.0312x over previous
import functools

import jax
import jax.numpy as jnp
from jax import lax
from jax.experimental import pallas as pl
from jax.experimental.pallas import tpu as pltpu
from jax.experimental.pallas import tpu_sc as plsc

D_MODEL = 128
BATCH = 16384

_info = plsc.get_sparse_core_info()
_NC, _NS = _info.num_cores, _info.num_subcores
_NW = _NC * _NS
_B_PER_W = BATCH // _NW

_mesh = plsc.VectorSubcoreMesh(core_axis_name="c", subcore_axis_name="s")

_NCHUNK = 2
_CH = _B_PER_W // _NCHUNK


@functools.partial(
    pl.kernel,
    mesh=_mesh,
    out_type=(
        jax.ShapeDtypeStruct((BATCH, D_MODEL), jnp.float32),
        jax.ShapeDtypeStruct((BATCH,), jnp.float32),
    ),
    scratch_types=[
        pltpu.VMEM((_B_PER_W,), jnp.int32),
        pltpu.VMEM((_B_PER_W, D_MODEL), jnp.float32),
        pltpu.VMEM((_B_PER_W,), jnp.float32),
    ]
    + [pltpu.SemaphoreType.DMA] * (2 * _NCHUNK + 2),
)
def _gather_rows(weight_hbm, bias_hbm, idx_hbm, out_w_hbm, out_b_hbm,
                 idx_v, rows_v, bvals_v, *sems):
    wid = lax.axis_index("s") * _NC + lax.axis_index("c")
    base = wid * _B_PER_W
    pltpu.sync_copy(idx_hbm.at[pl.ds(base, _B_PER_W)], idx_v)
    cb = pltpu.async_copy(bias_hbm.at[idx_v], bvals_v, sems[2 * _NCHUNK])
    gathers = [
        pltpu.async_copy(
            weight_hbm.at[idx_v.at[pl.ds(c * _CH, _CH)]],
            rows_v.at[pl.ds(c * _CH, _CH)],
            sems[c],
        )
        for c in range(_NCHUNK)
    ]
    writes = []
    for c in range(_NCHUNK):
        gathers[c].wait()
        writes.append(
            pltpu.async_copy(
                rows_v.at[pl.ds(c * _CH, _CH)],
                out_w_hbm.at[pl.ds(base + c * _CH, _CH)],
                sems[_NCHUNK + c],
            )
        )
    cb.wait()
    writes.append(
        pltpu.async_copy(
            bvals_v, out_b_hbm.at[pl.ds(base, _B_PER_W)], sems[2 * _NCHUNK + 1]
        )
    )
    for w in writes:
        w.wait()


def kernel(coords, voxel_indices, weight, bias):
    del coords
    idx = voxel_indices.astype(jnp.int32)
    return _gather_rows(weight, bias, idx)


# speedup vs baseline: 1.0357x; 1.0043x over previous
import functools

import jax
import jax.numpy as jnp
from jax import lax
from jax.experimental import pallas as pl
from jax.experimental.pallas import tpu as pltpu
from jax.experimental.pallas import tpu_sc as plsc

D_MODEL = 128
BATCH = 16384

_info = plsc.get_sparse_core_info()
_NC, _NS = _info.num_cores, _info.num_subcores
_NW = _NC * _NS
_B_PER_W = BATCH // _NW

_mesh = plsc.VectorSubcoreMesh(core_axis_name="c", subcore_axis_name="s")

_NCHUNK = 4
_CH = _B_PER_W // _NCHUNK


@functools.partial(
    pl.kernel,
    mesh=_mesh,
    out_type=(
        jax.ShapeDtypeStruct((BATCH, D_MODEL), jnp.float32),
        jax.ShapeDtypeStruct((BATCH,), jnp.float32),
    ),
    scratch_types=[
        pltpu.VMEM((_B_PER_W,), jnp.int32),
        pltpu.VMEM((_B_PER_W, D_MODEL), jnp.float32),
        pltpu.VMEM((_B_PER_W,), jnp.float32),
    ]
    + [pltpu.SemaphoreType.DMA] * (2 * _NCHUNK + 2),
)
def _gather_rows(weight_hbm, bias_hbm, idx_hbm, out_w_hbm, out_b_hbm,
                 idx_v, rows_v, bvals_v, *sems):
    wid = lax.axis_index("s") * _NC + lax.axis_index("c")
    base = wid * _B_PER_W
    pltpu.sync_copy(idx_hbm.at[pl.ds(base, _B_PER_W)], idx_v)
    cb = pltpu.async_copy(bias_hbm.at[idx_v], bvals_v, sems[2 * _NCHUNK])
    gathers = [
        pltpu.async_copy(
            weight_hbm.at[idx_v.at[pl.ds(c * _CH, _CH)]],
            rows_v.at[pl.ds(c * _CH, _CH)],
            sems[c],
        )
        for c in range(_NCHUNK)
    ]
    writes = []
    for c in range(_NCHUNK):
        gathers[c].wait()
        writes.append(
            pltpu.async_copy(
                rows_v.at[pl.ds(c * _CH, _CH)],
                out_w_hbm.at[pl.ds(base + c * _CH, _CH)],
                sems[_NCHUNK + c],
            )
        )
    cb.wait()
    writes.append(
        pltpu.async_copy(
            bvals_v, out_b_hbm.at[pl.ds(base, _B_PER_W)], sems[2 * _NCHUNK + 1]
        )
    )
    for w in writes:
        w.wait()


def kernel(coords, voxel_indices, weight, bias):
    del coords
    idx = voxel_indices.astype(jnp.int32)
    return _gather_rows(weight, bias, idx)


# speedup vs baseline: 1.0390x; 1.0032x over previous
import functools

import jax
import jax.numpy as jnp
from jax import lax
from jax.experimental import pallas as pl
from jax.experimental.pallas import tpu as pltpu
from jax.experimental.pallas import tpu_sc as plsc

D_MODEL = 128
BATCH = 16384

_info = plsc.get_sparse_core_info()
_NC, _NS = _info.num_cores, _info.num_subcores
_NW = _NC * _NS
_B_PER_W = BATCH // _NW

_mesh = plsc.VectorSubcoreMesh(core_axis_name="c", subcore_axis_name="s")

_NCHUNK = 2
_CH = _B_PER_W // _NCHUNK


@functools.partial(
    pl.kernel,
    mesh=_mesh,
    out_type=(
        jax.ShapeDtypeStruct((BATCH, D_MODEL), jnp.float32),
        jax.ShapeDtypeStruct((BATCH,), jnp.float32),
    ),
    scratch_types=[
        pltpu.VMEM((_B_PER_W,), jnp.int32),
        pltpu.VMEM((_B_PER_W, D_MODEL), jnp.float32),
        pltpu.VMEM((_B_PER_W,), jnp.float32),
    ]
    + [pltpu.SemaphoreType.DMA] * (2 * _NCHUNK + 2),
)
def _gather_rows(weight_hbm, bias_hbm, idx_hbm, out_w_hbm, out_b_hbm,
                 idx_v, rows_v, bvals_v, *sems):
    wid = lax.axis_index("s") * _NC + lax.axis_index("c")
    base = wid * _B_PER_W
    pltpu.sync_copy(idx_hbm.at[pl.ds(base, _B_PER_W)], idx_v)
    cb = pltpu.async_copy(bias_hbm.at[idx_v], bvals_v, sems[2 * _NCHUNK])
    gathers = [
        pltpu.async_copy(
            weight_hbm.at[idx_v.at[pl.ds(c * _CH, _CH)]],
            rows_v.at[pl.ds(c * _CH, _CH)],
            sems[c],
        )
        for c in range(_NCHUNK)
    ]
    writes = []
    for c in range(_NCHUNK):
        gathers[c].wait()
        writes.append(
            pltpu.async_copy(
                rows_v.at[pl.ds(c * _CH, _CH)],
                out_w_hbm.at[pl.ds(base + c * _CH, _CH)],
                sems[_NCHUNK + c],
            )
        )
    cb.wait()
    writes.append(
        pltpu.async_copy(
            bvals_v, out_b_hbm.at[pl.ds(base, _B_PER_W)], sems[2 * _NCHUNK + 1]
        )
    )
    for w in writes:
        w.wait()


def kernel(coords, voxel_indices, weight, bias):
    del coords
    idx = voxel_indices.astype(jnp.int32)
    return _gather_rows(weight, bias, idx)
